# Initial kernel scaffold; baseline (speedup 1.0000x reference)
import jax
import jax.numpy as jnp
from jax.experimental import pallas as pl


def kernel(embs, seq_lengths, pos_table):
    raise NotImplementedError("write your pallas kernel here")


# speedup vs baseline: 2.9261x; 2.9261x over previous
import functools

import jax
import jax.numpy as jnp
from jax import lax
from jax.experimental import pallas as pl
from jax.experimental.pallas import tpu as pltpu
from jax.experimental.pallas import tpu_sc as plsc

_LANES = 16


def _make_sc_kernel(B, L, D, n_workers, chunk_rows):
    rows_per_worker = B // n_workers
    n_chunks = rows_per_worker // chunk_rows
    vregs_per_pos = D // _LANES

    mesh = plsc.VectorSubcoreMesh(core_axis_name="c", subcore_axis_name="s")

    @functools.partial(
        pl.kernel,
        out_type=jax.ShapeDtypeStruct((B, L, D), jnp.float32),
        mesh=mesh,
        scratch_types=[
            pltpu.VMEM((L, D), jnp.float32),
            pltpu.VMEM((rows_per_worker + _LANES,), jnp.int32),
            pltpu.VMEM((chunk_rows, L, D), jnp.float32),
        ],
    )
    def sc_kernel(embs_hbm, slen_hbm, ptab_hbm, out_hbm, ptab_v, slen_v, buf_v):
        wid = lax.axis_index("s") * 2 + lax.axis_index("c")
        base = wid * rows_per_worker

        pltpu.sync_copy(ptab_hbm, ptab_v)
        pltpu.sync_copy(
            slen_hbm.at[pl.ds(base, rows_per_worker)],
            slen_v.at[pl.ds(0, rows_per_worker)],
        )

        def chunk_body(c, carry):
            row0 = base + c * chunk_rows
            pltpu.sync_copy(embs_hbm.at[pl.ds(row0, chunk_rows)], buf_v)
            for r in range(chunk_rows):
                n = slen_v[pl.ds(c * chunk_rows + r, _LANES)][0]

                def pos_body(j, inner):
                    for q in range(vregs_per_pos):
                        sl = pl.ds(q * _LANES, _LANES)
                        plsc.addupdate(buf_v.at[r, j, sl], ptab_v[j, sl])
                    return inner

                lax.fori_loop(0, n, pos_body, 0)
            pltpu.sync_copy(buf_v, out_hbm.at[pl.ds(row0, chunk_rows)])
            return carry

        lax.fori_loop(0, n_chunks, chunk_body, 0)

    return sc_kernel


def kernel(embs, seq_lengths, pos_table):
    B, L, D = embs.shape
    sc = _make_sc_kernel(B, L, D, n_workers=32, chunk_rows=4)
    return sc(embs, seq_lengths.astype(jnp.int32), pos_table[1 : L + 1])


# speedup vs baseline: 3.3131x; 1.1323x over previous
import functools

import jax
import jax.numpy as jnp
from jax import lax
from jax.experimental import pallas as pl
from jax.experimental.pallas import tpu as pltpu
from jax.experimental.pallas import tpu_sc as plsc

_LANES = 16


def _make_sc_kernel(B, L, D, n_workers, nbuf):
    rows_per_worker = B // n_workers
    n_groups = rows_per_worker // nbuf
    vregs_per_pos = D // _LANES

    mesh = plsc.VectorSubcoreMesh(core_axis_name="c", subcore_axis_name="s")

    @functools.partial(
        pl.kernel,
        out_type=jax.ShapeDtypeStruct((B, L, D), jnp.float32),
        mesh=mesh,
        scratch_types=[
            pltpu.VMEM((L, D), jnp.float32),
            pltpu.VMEM((rows_per_worker + _LANES,), jnp.int32),
            pltpu.VMEM((nbuf, L, D), jnp.float32),
        ]
        + [pltpu.SemaphoreType.DMA] * (2 * nbuf),
    )
    def sc_kernel(embs_hbm, slen_hbm, ptab_hbm, out_hbm, ptab_v, slen_v, buf_v, *sems):
        in_sems = sems[:nbuf]
        out_sems = sems[nbuf:]
        wid = lax.axis_index("s") * 2 + lax.axis_index("c")
        base = wid * rows_per_worker

        pltpu.sync_copy(ptab_hbm, ptab_v)
        pltpu.sync_copy(
            slen_hbm.at[pl.ds(base, rows_per_worker)],
            slen_v.at[pl.ds(0, rows_per_worker)],
        )

        def in_copy(c, b):
            return pltpu.make_async_copy(
                embs_hbm.at[base + c], buf_v.at[b], in_sems[b]
            )

        def out_copy(c, b):
            return pltpu.make_async_copy(
                buf_v.at[b], out_hbm.at[base + c], out_sems[b]
            )

        def group_body(g, carry):
            c0 = g * nbuf
            for b in range(nbuf):

                @pl.when(g > 0)
                def _wait_prev_out(b=b):
                    out_copy((g - 1) * nbuf + b, b).wait()

                in_copy(c0 + b, b).start()

            for b in range(nbuf):
                c = c0 + b
                in_copy(c, b).wait()
                n = slen_v[pl.ds(c, _LANES)][0]

                def pos_body(j, inner, _b=b):
                    for q in range(vregs_per_pos):
                        sl = pl.ds(q * _LANES, _LANES)
                        plsc.addupdate(buf_v.at[_b, j, sl], ptab_v[j, sl])
                    return inner

                lax.fori_loop(0, n, pos_body, 0)
                out_copy(c, b).start()
            return carry

        lax.fori_loop(0, n_groups, group_body, 0)
        for b in range(nbuf):
            out_copy(rows_per_worker - nbuf + b, b).wait()

    return sc_kernel


def kernel(embs, seq_lengths, pos_table):
    B, L, D = embs.shape
    sc = _make_sc_kernel(B, L, D, n_workers=32, nbuf=4)
    return sc(embs, seq_lengths.astype(jnp.int32), pos_table[1 : L + 1])
